# Initial kernel scaffold; baseline (speedup 1.0000x reference)
import jax
import jax.numpy as jnp
from jax.experimental import pallas as pl


def kernel(x, W_diff, b_diff, W_self, b_self, bias):
    raise NotImplementedError("write your pallas kernel here")


# speedup vs baseline: 12.1502x; 12.1502x over previous
import jax
import jax.numpy as jnp
from jax.experimental import pallas as pl

_K = 8
_BR = 256


def _knn_agg_kernel(xb_ref, xr_ref, wd_ref, wc_ref, bb_ref, out_ref):
    X = xb_ref[0]
    Xr = xr_ref[0]
    n = X.shape[0]

    rr = jnp.sum(Xr * Xr, axis=1, keepdims=True)
    cc = jnp.sum(X * X, axis=1)[None, :]
    mul = jax.lax.dot_general(
        Xr, X, (((1,), (1,)), ((), ())), preferred_element_type=jnp.float32
    )
    dist = rr - 2.0 * mul + cc

    iota = jax.lax.broadcasted_iota(jnp.int32, dist.shape, 1)

    def body(_, carry):
        d, s = carry
        m = jnp.min(d, axis=1, keepdims=True)
        idx = jnp.min(jnp.where(d == m, iota, n), axis=1, keepdims=True)
        sel = iota == idx
        s = s + sel.astype(jnp.float32)
        d = jnp.where(sel, jnp.inf, d)
        return d, s

    _, S = jax.lax.fori_loop(
        0, _K, body, (dist, jnp.zeros(dist.shape, jnp.float32))
    )

    nb_sum = jax.lax.dot_general(
        S, X, (((1,), (0,)), ((), ())), preferred_element_type=jnp.float32
    )
    out = (
        jnp.dot(nb_sum * (1.0 / _K), wd_ref[...],
                preferred_element_type=jnp.float32)
        + jnp.dot(Xr, wc_ref[...], preferred_element_type=jnp.float32)
        + bb_ref[...]
    )
    out_ref[0] = out


def kernel(x, W_diff, b_diff, W_self, b_self, bias):
    b, f, h, w = x.shape
    n = h * w
    xf = jnp.transpose(x, (0, 2, 3, 1)).reshape(b, n, f)
    wd_t = W_diff.T
    wc_t = (W_self - W_diff).T
    b_all = (b_diff + b_self + bias).reshape(1, f)

    out = pl.pallas_call(
        _knn_agg_kernel,
        grid=(b, n // _BR),
        in_specs=[
            pl.BlockSpec((1, n, f), lambda bi, ri: (bi, 0, 0)),
            pl.BlockSpec((1, _BR, f), lambda bi, ri: (bi, ri, 0)),
            pl.BlockSpec((f, f), lambda bi, ri: (0, 0)),
            pl.BlockSpec((f, f), lambda bi, ri: (0, 0)),
            pl.BlockSpec((1, f), lambda bi, ri: (0, 0)),
        ],
        out_specs=pl.BlockSpec((1, _BR, f), lambda bi, ri: (bi, ri, 0)),
        out_shape=jax.ShapeDtypeStruct((b, n, f), jnp.float32),
    )(xf, xf, wd_t, wc_t, b_all)

    return jnp.transpose(out.reshape(b, h, w, f), (0, 3, 1, 2))


# speedup vs baseline: 16.9981x; 1.3990x over previous
import jax
import jax.numpy as jnp
from jax.experimental import pallas as pl

_K = 8
_BR = 256


def _knn_agg_kernel(xb_ref, xr_ref, wd_ref, wc_ref, bb_ref, out_ref):
    X = xb_ref[0]
    Xr = xr_ref[0]
    n = X.shape[0]

    rr = jnp.sum(Xr * Xr, axis=1, keepdims=True)
    cc = jnp.sum(X * X, axis=1)[None, :]
    mul = jax.lax.dot_general(
        Xr, X, (((1,), (1,)), ((), ())), preferred_element_type=jnp.float32
    )
    dist = rr - 2.0 * mul + cc

    iota = jax.lax.broadcasted_iota(jnp.int32, dist.shape, 1)

    def body(_, d):
        idx = jnp.argmin(d, axis=1)[:, None]
        return jnp.where(iota == idx, jnp.inf, d)

    d = jax.lax.fori_loop(0, _K, body, dist)
    S = (d == jnp.inf).astype(jnp.float32)

    nb_sum = jax.lax.dot_general(
        S, X, (((1,), (0,)), ((), ())), preferred_element_type=jnp.float32
    )
    out = (
        jnp.dot(nb_sum * (1.0 / _K), wd_ref[...],
                preferred_element_type=jnp.float32)
        + jnp.dot(Xr, wc_ref[...], preferred_element_type=jnp.float32)
        + bb_ref[...]
    )
    out_ref[0] = out


def kernel(x, W_diff, b_diff, W_self, b_self, bias):
    b, f, h, w = x.shape
    n = h * w
    xf = jnp.transpose(x, (0, 2, 3, 1)).reshape(b, n, f)
    wd_t = W_diff.T
    wc_t = (W_self - W_diff).T
    b_all = (b_diff + b_self + bias).reshape(1, f)

    out = pl.pallas_call(
        _knn_agg_kernel,
        grid=(b, n // _BR),
        in_specs=[
            pl.BlockSpec((1, n, f), lambda bi, ri: (bi, 0, 0)),
            pl.BlockSpec((1, _BR, f), lambda bi, ri: (bi, ri, 0)),
            pl.BlockSpec((f, f), lambda bi, ri: (0, 0)),
            pl.BlockSpec((f, f), lambda bi, ri: (0, 0)),
            pl.BlockSpec((1, f), lambda bi, ri: (0, 0)),
        ],
        out_specs=pl.BlockSpec((1, _BR, f), lambda bi, ri: (bi, ri, 0)),
        out_shape=jax.ShapeDtypeStruct((b, n, f), jnp.float32),
    )(xf, xf, wd_t, wc_t, b_all)

    return jnp.transpose(out.reshape(b, h, w, f), (0, 3, 1, 2))


# speedup vs baseline: 34.2090x; 2.0125x over previous
import jax
import jax.numpy as jnp
from jax.experimental import pallas as pl

_K = 8
_BR = 256


def _knn_agg_kernel(xb_ref, xr_ref, wd_ref, wc_ref, bb_ref, out_ref):
    X = xb_ref[0]
    Xr = xr_ref[0]
    n = X.shape[0]

    rr = jnp.sum(Xr * Xr, axis=1, keepdims=True)
    cc = jnp.sum(X * X, axis=1)[None, :]
    mul = jax.lax.dot_general(
        Xr, X, (((1,), (1,)), ((), ())), preferred_element_type=jnp.float32
    )
    dist = rr - 2.0 * mul + cc

    iota = jax.lax.broadcasted_iota(jnp.int32, dist.shape, 1)

    d = dist
    for _ in range(_K):
        idx = jnp.argmin(d, axis=1)[:, None]
        d = jnp.where(iota == idx, jnp.inf, d)
    S = (d == jnp.inf).astype(jnp.float32)

    nb_sum = jax.lax.dot_general(
        S, X, (((1,), (0,)), ((), ())), preferred_element_type=jnp.float32
    )
    out = (
        jnp.dot(nb_sum * (1.0 / _K), wd_ref[...],
                preferred_element_type=jnp.float32)
        + jnp.dot(Xr, wc_ref[...], preferred_element_type=jnp.float32)
        + bb_ref[...]
    )
    out_ref[0] = out


def kernel(x, W_diff, b_diff, W_self, b_self, bias):
    b, f, h, w = x.shape
    n = h * w
    xf = jnp.transpose(x, (0, 2, 3, 1)).reshape(b, n, f)
    wd_t = W_diff.T
    wc_t = (W_self - W_diff).T
    b_all = (b_diff + b_self + bias).reshape(1, f)

    out = pl.pallas_call(
        _knn_agg_kernel,
        grid=(b, n // _BR),
        in_specs=[
            pl.BlockSpec((1, n, f), lambda bi, ri: (bi, 0, 0)),
            pl.BlockSpec((1, _BR, f), lambda bi, ri: (bi, ri, 0)),
            pl.BlockSpec((f, f), lambda bi, ri: (0, 0)),
            pl.BlockSpec((f, f), lambda bi, ri: (0, 0)),
            pl.BlockSpec((1, f), lambda bi, ri: (0, 0)),
        ],
        out_specs=pl.BlockSpec((1, _BR, f), lambda bi, ri: (bi, ri, 0)),
        out_shape=jax.ShapeDtypeStruct((b, n, f), jnp.float32),
    )(xf, xf, wd_t, wc_t, b_all)

    return jnp.transpose(out.reshape(b, h, w, f), (0, 3, 1, 2))
